# Initial kernel scaffold; baseline (speedup 1.0000x reference)
import jax
import jax.numpy as jnp
from jax.experimental import pallas as pl


def kernel(node_u, edge_index, edge_attr, gamma, params):
    raise NotImplementedError("write your pallas kernel here")


# speedup vs baseline: 3.2938x; 3.2938x over previous
import functools

import jax
import jax.numpy as jnp
from jax import lax
from jax.experimental import pallas as pl
from jax.experimental.pallas import tpu as pltpu
from jax.experimental.pallas import tpu_sc as plsc

N_NODES = 50000
E_EDGES = 800000
DT_MAX = 0.015

NC, NS = 2, 16
NW = NC * NS
E_PAD = 819200
EPT = E_PAD // NW
GCH = 512
NCH = EPT // GCH
IDXW = 128

SROWS = 2 * E_PAD // IDXW
IROWS_T = SROWS // NW
KB = 16
SCH = IROWS_T // KB

BN = 5000
BE = 2048

_SQRT1_2 = 0.7071067811865476


def _gelu(x):
    return x * 0.5 * (1.0 + lax.erf(x * _SQRT1_2))


def _node_body(u_ref, wn1, bn1, wn2, bn2, wg1, bg1, wg2, bg2, g_ref):
    x = u_ref[...]
    h = _gelu(jnp.dot(x, wn1[...], preferred_element_type=jnp.float32) + bn1[...])
    h = jnp.dot(h, wn2[...], preferred_element_type=jnp.float32) + bn2[...]
    t = _gelu(jnp.dot(h, wg1[...], preferred_element_type=jnp.float32) + bg1[...])
    g_ref[...] = jnp.dot(t, wg2[...], preferred_element_type=jnp.float32) + bg2[...]


def _gather_body(g_hbm, src_hbm, dst_hbm, vg_hbm, idx_a, idx_b, buf_a, buf_b, sem):
    wid = lax.axis_index("s") * NC + lax.axis_index("c")
    base = wid * EPT

    def chunk(cc, carry):
        cb = base + cc * GCH
        pltpu.sync_copy(src_hbm.at[pl.ds(cb, GCH)], idx_a)
        pltpu.sync_copy(dst_hbm.at[pl.ds(cb, GCH)], idx_b)
        cps = []
        for k in range(GCH // IDXW):
            sl = pl.ds(k * IDXW, IDXW)
            cps.append(pltpu.async_copy(g_hbm.at[idx_a.at[sl]], buf_a.at[sl], sem))
            cps.append(pltpu.async_copy(g_hbm.at[idx_b.at[sl]], buf_b.at[sl], sem))
        for cp in cps:
            cp.wait()

        def add_row(i, c2):
            for j in range(4):
                jl = pl.ds(j * 16, 16)
                buf_a[i, jl] = buf_a[i, jl] + buf_b[i, jl]
            return c2

        lax.fori_loop(0, GCH, add_row, 0)
        pltpu.sync_copy(buf_a, vg_hbm.at[pl.ds(cb, GCH)])
        return carry

    lax.fori_loop(0, NCH, chunk, 0)


def _edge_body(vg_ref, ea_ref, src_ref, dst_ref,
               a_pe, b_pe1, w_pe2, b_pe2,
               wv, we, b_m1, wm2, b_m2,
               wh1, bh1, wh2, bh2,
               raw2_ref, ssum_ref, scnt_ref):
    pid = pl.program_id(0)
    vg = vg_ref[...]
    ea = ea_ref[...]
    dx = ea[:, 0:1]
    dy = ea[:, 1:2]
    r = ea[:, 2:3]

    eps_h = _gelu(r * a_pe[...] + b_pe1[...])
    eps = jnp.dot(eps_h, w_pe2[...], preferred_element_type=jnp.float32) + b_pe2[...]
    m1 = _gelu(jnp.dot(vg, wv[...], preferred_element_type=jnp.float32)
               + jnp.dot(eps, we[...], preferred_element_type=jnp.float32)
               + b_m1[...])
    m = jnp.dot(m1, wm2[...], preferred_element_type=jnp.float32) + b_m2[...]
    hh = _gelu(jnp.dot(m, wh1[...], preferred_element_type=jnp.float32) + bh1[...])
    a = jnp.dot(hh, wh2[...], preferred_element_type=jnp.float32) + bh2[...]

    inv = 1.0 / (r + 1e-12)
    nx = dx * inv
    ny = dy * inv
    n2 = nx * nx + ny * ny
    maskf = (src_ref[...] < dst_ref[...]).astype(jnp.float32)
    a0 = a[:, 0:1]
    a1 = a[:, 1:2]
    a2 = a[:, 2:3]
    a3 = a[:, 3:4]
    raw = jnp.concatenate(
        [a0 * n2 * r, a1 * n2 * r, (a2 * nx - a3 * ny) * r, (a2 * ny + a3 * nx) * r],
        axis=1) * maskf
    raw2_ref[0] = raw
    raw2_ref[1] = -raw

    ps = jnp.sum(r * maskf).reshape(1, 1)
    pc = jnp.sum(maskf).reshape(1, 1)

    @pl.when(pid == 0)
    def _():
        ssum_ref[...] = jnp.zeros_like(ssum_ref)
        scnt_ref[...] = jnp.zeros_like(scnt_ref)

    ssum_ref[...] += ps
    scnt_ref[...] += pc


def _scatter_body(raw_hbm, idx_hbm, zeros_hbm, out_hbm, acc, vbuf, ibuf):
    c = lax.axis_index("c")
    s = lax.axis_index("s")
    wid = s * NC + c

    @pl.when(s == 0)
    def _():
        pltpu.sync_copy(zeros_hbm, acc)

    plsc.subcore_barrier()

    def chunk(cc, carry):
        ir0 = wid * IROWS_T + cc * KB
        vb0 = ir0 * IDXW
        pltpu.sync_copy(idx_hbm.at[pl.ds(ir0, KB)], ibuf)
        pltpu.sync_copy(raw_hbm.at[pl.ds(vb0, KB * IDXW)], vbuf)
        for j in range(KB):
            pltpu.sync_copy(vbuf.at[pl.ds(j * IDXW, IDXW)],
                            acc.at[ibuf.at[j]], add=True)
        return carry

    lax.fori_loop(0, SCH, chunk, 0)
    plsc.subcore_barrier()

    @pl.when(s == 0)
    def _():
        pltpu.sync_copy(acc, out_hbm.at[c])


def _combine_body(u_ref, s_ref, scale_ref, out_ref):
    u = u_ref[...]
    d = (s_ref[0] + s_ref[1]) * scale_ref[0, 0]
    zero = jnp.zeros_like(d[:, 0:1])
    out_ref[...] = u + jnp.concatenate([d[:, 0:2], zero, d[:, 2:4]], axis=1)


def _full(shape):
    return pl.BlockSpec(shape, lambda i: tuple(0 for _ in shape))


def kernel(node_u, edge_index, edge_attr, gamma, params):
    p = params
    f32 = jnp.float32

    wn1 = p["phi_node_w1"].T
    bn1 = p["phi_node_b1"].reshape(1, 64)
    wn2 = p["phi_node_w2"].T
    bn2 = p["phi_node_b2"].reshape(1, 64)
    wg1 = jnp.concatenate([p["phi1_w1"], p["phi2_w1"]], axis=0).T
    bg1 = jnp.concatenate([p["phi1_b1"], p["phi2_b1"]]).reshape(1, 64)
    wg2 = jnp.concatenate([p["phi1_w2"].T, p["phi2_w2"].T], axis=0)
    bg2 = (p["phi1_b2"] + p["phi2_b2"]).reshape(1, 64)

    a_pe = p["phi_edge_w1"][:, 0].reshape(1, 32)
    b_pe1 = p["phi_edge_b1"].reshape(1, 32)
    w_pe2 = p["phi_edge_w2"].T
    b_pe2 = p["phi_edge_b2"].reshape(1, 32)
    wv = p["phi_msg_w1"][:, :64].T
    we = p["phi_msg_w1"][:, 64:].T
    b_m1 = p["phi_msg_b1"].reshape(1, 128)
    wm2 = p["phi_msg_w2"].T
    b_m2 = p["phi_msg_b2"].reshape(1, 64)
    wh1 = jnp.concatenate([p["psi_rho_w1"], p["psi_e_w1"], p["psi_rhou_w1"]],
                          axis=0).T
    bh1 = jnp.concatenate([p["psi_rho_b1"], p["psi_e_b1"],
                           p["psi_rhou_b1"]]).reshape(1, 192)
    wh2 = jnp.zeros((192, 4), f32)
    wh2 = wh2.at[0:64, 0].set(p["psi_rho_w2"][0])
    wh2 = wh2.at[64:128, 1].set(p["psi_e_w2"][0])
    wh2 = wh2.at[128:192, 2:4].set(p["psi_rhou_w2"].T)
    bh2 = jnp.concatenate([p["psi_rho_b2"], p["psi_e_b2"],
                           p["psi_rhou_b2"]]).reshape(1, 4)

    pad = E_PAD - E_EDGES
    ei_pad = jnp.pad(edge_index, ((0, 0), (0, pad)))
    ea_pad = jnp.pad(edge_attr, ((0, pad), (0, 0)))
    src = ei_pad[0]
    dst = ei_pad[1]
    src2d = src.reshape(E_PAD, 1)
    dst2d = dst.reshape(E_PAD, 1)
    idx2d = ei_pad.reshape(SROWS, IDXW)
    zeros_n4 = jnp.zeros((N_NODES, 4), f32)

    g = pl.pallas_call(
        _node_body,
        grid=(N_NODES // BN,),
        in_specs=[
            pl.BlockSpec((BN, 5), lambda i: (i, 0)),
            _full((5, 64)), _full((1, 64)), _full((64, 64)), _full((1, 64)),
            _full((64, 64)), _full((1, 64)), _full((64, 64)), _full((1, 64)),
        ],
        out_specs=pl.BlockSpec((BN, 64), lambda i: (i, 0)),
        out_shape=jax.ShapeDtypeStruct((N_NODES, 64), f32),
    )(node_u, wn1, bn1, wn2, bn2, wg1, bg1, wg2, bg2)

    mesh = plsc.VectorSubcoreMesh(core_axis_name="c", subcore_axis_name="s",
                                  num_cores=NC, num_subcores=NS)
    vg = pl.kernel(
        _gather_body,
        jax.ShapeDtypeStruct((E_PAD, 64), f32),
        mesh=mesh,
        compiler_params=pltpu.CompilerParams(use_tc_tiling_on_sc=False),
        scratch_types=[
            pltpu.VMEM((GCH,), jnp.int32),
            pltpu.VMEM((GCH,), jnp.int32),
            pltpu.VMEM((GCH, 64), f32),
            pltpu.VMEM((GCH, 64), f32),
            pltpu.SemaphoreType.DMA,
        ],
    )(g, src, dst)

    raw2, ssum, scnt = pl.pallas_call(
        _edge_body,
        grid=(E_PAD // BE,),
        in_specs=[
            pl.BlockSpec((BE, 64), lambda i: (i, 0)),
            pl.BlockSpec((BE, 3), lambda i: (i, 0)),
            pl.BlockSpec((BE, 1), lambda i: (i, 0)),
            pl.BlockSpec((BE, 1), lambda i: (i, 0)),
            _full((1, 32)), _full((1, 32)), _full((32, 32)), _full((1, 32)),
            _full((64, 128)), _full((32, 128)), _full((1, 128)),
            _full((128, 64)), _full((1, 64)),
            _full((64, 192)), _full((1, 192)), _full((192, 4)), _full((1, 4)),
        ],
        out_specs=[
            pl.BlockSpec((2, BE, 4), lambda i: (0, i, 0)),
            pl.BlockSpec((1, 1), lambda i: (0, 0)),
            pl.BlockSpec((1, 1), lambda i: (0, 0)),
        ],
        out_shape=[
            jax.ShapeDtypeStruct((2, E_PAD, 4), f32),
            jax.ShapeDtypeStruct((1, 1), f32),
            jax.ShapeDtypeStruct((1, 1), f32),
        ],
    )(vg, ea_pad, src2d, dst2d,
      a_pe, b_pe1, w_pe2, b_pe2, wv, we, b_m1, wm2, b_m2, wh1, bh1, wh2, bh2)

    raw_flat = raw2.reshape(2 * E_PAD, 4)
    s_part = pl.kernel(
        _scatter_body,
        jax.ShapeDtypeStruct((NC, N_NODES, 4), f32),
        mesh=mesh,
        compiler_params=pltpu.CompilerParams(use_tc_tiling_on_sc=False),
        scratch_types=[
            pltpu.VMEM_SHARED((N_NODES, 4), f32),
            pltpu.VMEM((KB * IDXW, 4), f32),
            pltpu.VMEM((KB, IDXW), jnp.int32),
        ],
    )(raw_flat, idx2d, zeros_n4)

    dt = DT_MAX * jax.nn.sigmoid(p["s"])
    dx_est = ssum[0, 0] / scnt[0, 0]
    scale = (-dt / (dx_est * dx_est)).reshape(1, 1)

    out = pl.pallas_call(
        _combine_body,
        grid=(N_NODES // BN,),
        in_specs=[
            pl.BlockSpec((BN, 5), lambda i: (i, 0)),
            pl.BlockSpec((NC, BN, 4), lambda i: (0, i, 0)),
            pl.BlockSpec(memory_space=pltpu.SMEM),
        ],
        out_specs=pl.BlockSpec((BN, 5), lambda i: (i, 0)),
        out_shape=jax.ShapeDtypeStruct((N_NODES, 5), f32),
    )(node_u, s_part, scale)

    return out


# speedup vs baseline: 9.9750x; 3.0284x over previous
import jax
import jax.numpy as jnp
from jax import lax
from jax.experimental import pallas as pl
from jax.experimental.pallas import tpu as pltpu
from jax.experimental.pallas import tpu_sc as plsc

N_NODES = 50000
E_EDGES = 800000
DT_MAX = 0.015

NC, NS = 2, 16
NW = NC * NS
E_PAD = 819200
NSPLIT = 2
E_H = E_PAD // NSPLIT
EPT = E_H // NW
GCH = 256
NCH = EPT // GCH
IDXW = 128

K8 = E_H // IDXW
KPT = K8 // NW
KR = 4
SCH = KPT // KR

BN = 5000
BE = 2048

_SQRT1_2 = 0.7071067811865476


def _gelu(x):
    return x * 0.5 * (1.0 + lax.erf(x * _SQRT1_2))


def _node_body(u_ref, wn1, bn1, wn2, bn2, wg1, bg1, wg2, bg2, g_ref):
    x = u_ref[...]
    h = _gelu(jnp.dot(x, wn1[...], preferred_element_type=jnp.float32) + bn1[...])
    h = jnp.dot(h, wn2[...], preferred_element_type=jnp.float32) + bn2[...]
    t = _gelu(jnp.dot(h, wg1[...], preferred_element_type=jnp.float32) + bg1[...])
    g = jnp.dot(t, wg2[...], preferred_element_type=jnp.float32) + bg2[...]
    g_ref[...] = jnp.concatenate([g, jnp.zeros_like(g)], axis=1).astype(jnp.bfloat16)


def _gather_body(g_hbm, src_hbm, dst_hbm, vg_hbm,
                 idx_a0, idx_b0, idx_a1, idx_b1,
                 buf_a0, buf_b0, buf_a1, buf_b1,
                 gsem0, gsem1, osem0, osem1):
    wid = lax.axis_index("s") * NC + lax.axis_index("c")
    base = wid * EPT
    sets = [(idx_a0, idx_b0, buf_a0, buf_b0, gsem0, osem0),
            (idx_a1, idx_b1, buf_a1, buf_b1, gsem1, osem1)]

    def fire(st, cc):
        idx_a, idx_b, buf_a, buf_b, gsem, _ = st
        cb = base + cc * GCH
        pltpu.sync_copy(src_hbm.at[pl.ds(cb, GCH)], idx_a)
        pltpu.sync_copy(dst_hbm.at[pl.ds(cb, GCH)], idx_b)
        for k in range(GCH // IDXW):
            sl = pl.ds(k * IDXW, IDXW)
            pltpu.async_copy(g_hbm.at[idx_a.at[sl]], buf_a.at[sl], gsem)
            pltpu.async_copy(g_hbm.at[idx_b.at[sl]], buf_b.at[sl], gsem)

    def finish(st, cc):
        idx_a, idx_b, buf_a, buf_b, gsem, osem = st
        for k in range(GCH // IDXW):
            sl = pl.ds(k * IDXW, IDXW)
            pltpu.make_async_copy(g_hbm.at[idx_a.at[sl]], buf_a.at[sl], gsem).wait()
            pltpu.make_async_copy(g_hbm.at[idx_b.at[sl]], buf_b.at[sl], gsem).wait()

        def add_row(i, c2):
            for j in range(2):
                jl = pl.ds(j * 32, 32)
                buf_a[i, jl] = buf_a[i, jl] + buf_b[i, jl]
            return c2

        lax.fori_loop(0, GCH, add_row, 0)
        pltpu.async_copy(buf_a, vg_hbm.at[pl.ds(base + cc * GCH, GCH)], osem)

    def drain_out(st):
        _, _, buf_a, _, _, osem = st
        pltpu.make_async_copy(buf_a, vg_hbm.at[pl.ds(base, GCH)], osem).wait()

    fire(sets[0], 0)

    def body(i, carry):
        @pl.when(i > 0)
        def _():
            drain_out(sets[1])
        fire(sets[1], 2 * i + 1)
        finish(sets[0], 2 * i)

        @pl.when(2 * i + 2 < NCH)
        def _():
            drain_out(sets[0])
            fire(sets[0], 2 * i + 2)
        finish(sets[1], 2 * i + 1)
        return carry

    lax.fori_loop(0, NCH // 2, body, 0)
    drain_out(sets[0])
    drain_out(sets[1])


def _edge_body(vg_ref, eaT_ref, ei_ref,
               a_pe, b_pe1, w_pe2, b_pe2,
               wv, we, b_m1, wm2, b_m2,
               wh1, bh1, wh2, bh2,
               raw_ref, ssum_ref, scnt_ref):
    pid = pl.program_id(0)
    vg = vg_ref[...]
    dxR = eaT_ref[0:1, :]
    dyR = eaT_ref[1:2, :]
    rT = eaT_ref[2:3, :]
    r = jnp.transpose(rT)

    bf = jnp.bfloat16
    eps_h = _gelu(r * a_pe[...] + b_pe1[...])
    eps = jnp.dot(eps_h.astype(bf), w_pe2[...],
                  preferred_element_type=jnp.float32) + b_pe2[...]
    m1 = _gelu(jnp.dot(vg, wv[...], preferred_element_type=jnp.float32)
               + jnp.dot(eps.astype(bf), we[...],
                         preferred_element_type=jnp.float32)
               + b_m1[...])
    m = jnp.dot(m1.astype(bf), wm2[...],
                preferred_element_type=jnp.float32) + b_m2[...]
    hh = _gelu(jnp.dot(m.astype(bf), wh1[...],
                       preferred_element_type=jnp.float32) + bh1[...])
    a = jnp.dot(hh.astype(bf), wh2[...],
                preferred_element_type=jnp.float32) + bh2[...]

    aT = jnp.transpose(a)
    invR = 1.0 / (rT + 1e-12)
    nxR = dxR * invR
    nyR = dyR * invR
    n2R = nxR * nxR + nyR * nyR
    maskR = (ei_ref[0:1, :] < ei_ref[1:2, :]).astype(jnp.float32)
    raw0 = aT[0:1, :] * n2R * rT
    raw1 = aT[1:2, :] * n2R * rT
    raw2 = (aT[2:3, :] * nxR - aT[3:4, :] * nyR) * rT
    raw3 = (aT[2:3, :] * nyR + aT[3:4, :] * nxR) * rT
    pos = jnp.concatenate([raw0, raw1, raw2, raw3], axis=0) * maskR
    pos8 = jnp.concatenate([pos, -pos], axis=0)
    for k in range(BE // IDXW):
        raw_ref[k] = pos8[:, k * IDXW:(k + 1) * IDXW]

    ps = jnp.sum(rT * maskR).reshape(1, 1)
    pc = jnp.sum(maskR).reshape(1, 1)

    @pl.when(pid == 0)
    def _():
        ssum_ref[...] = jnp.zeros_like(ssum_ref)
        scnt_ref[...] = jnp.zeros_like(scnt_ref)

    ssum_ref[...] += ps
    scnt_ref[...] += pc


def _scatter_body(raw_hbm, idx_hbm, zeros_hbm, out_hbm,
                  acc0, acc1, acc2, acc3, vbuf, ibuf, ssem):
    c = lax.axis_index("c")
    s = lax.axis_index("s")
    wid = s * NC + c
    accs = [acc0, acc1, acc2, acc3]

    @pl.when(s == 0)
    def _():
        for a in accs:
            pltpu.sync_copy(zeros_hbm, a)

    plsc.subcore_barrier()

    def chunk(cc, carry):
        k0 = wid * KPT + cc * KR
        pltpu.sync_copy(raw_hbm.at[pl.ds(k0, KR)], vbuf)
        pltpu.sync_copy(idx_hbm.at[:, pl.ds(k0, KR)], ibuf)

        def row(jj, c2):
            for d in range(2):
                for comp in range(4):
                    pltpu.async_copy(vbuf.at[jj, comp + 4 * d],
                                     accs[comp].at[ibuf.at[d, jj]], ssem,
                                     add=True)
            for d in range(2):
                for comp in range(4):
                    pltpu.make_async_copy(vbuf.at[jj, comp + 4 * d],
                                          accs[comp].at[ibuf.at[d, jj]],
                                          ssem).wait()
            return c2

        lax.fori_loop(0, KR, row, 0)
        return carry

    lax.fori_loop(0, SCH, chunk, 0)
    plsc.subcore_barrier()

    @pl.when(s == 0)
    def _():
        for comp in range(4):
            pltpu.sync_copy(accs[comp], out_hbm.at[c, comp])


def _combine_body(uT_ref, s0_ref, s1_ref, scale_ref, outT_ref):
    uT = uT_ref[...]
    sc = scale_ref[0, 0]
    d = (s0_ref[0] + s0_ref[1] + s1_ref[0] + s1_ref[1]) * sc
    outT_ref[...] = uT + jnp.concatenate(
        [d[0:2, :], jnp.zeros_like(d[0:1, :]), d[2:4, :]], axis=0)


def _full(shape):
    return pl.BlockSpec(shape, lambda i: tuple(0 for _ in shape))


def kernel(node_u, edge_index, edge_attr, gamma, params):
    p = params
    f32 = jnp.float32
    bf = jnp.bfloat16

    wn1 = p["phi_node_w1"].T
    bn1 = p["phi_node_b1"].reshape(1, 64)
    wn2 = p["phi_node_w2"].T
    bn2 = p["phi_node_b2"].reshape(1, 64)
    wg1 = jnp.concatenate([p["phi1_w1"], p["phi2_w1"]], axis=0).T
    bg1 = jnp.concatenate([p["phi1_b1"], p["phi2_b1"]]).reshape(1, 64)
    wg2 = jnp.concatenate([p["phi1_w2"].T, p["phi2_w2"].T], axis=0)
    bg2 = (p["phi1_b2"] + p["phi2_b2"]).reshape(1, 64)

    a_pe = p["phi_edge_w1"][:, 0].reshape(1, 32)
    b_pe1 = p["phi_edge_b1"].reshape(1, 32)
    w_pe2 = p["phi_edge_w2"].T.astype(bf)
    b_pe2 = p["phi_edge_b2"].reshape(1, 32)
    wv = jnp.concatenate([p["phi_msg_w1"][:, :64].T,
                          jnp.zeros((64, 128), f32)], axis=0).astype(bf)
    we = p["phi_msg_w1"][:, 64:].T.astype(bf)
    b_m1 = p["phi_msg_b1"].reshape(1, 128)
    wm2 = p["phi_msg_w2"].T.astype(bf)
    b_m2 = p["phi_msg_b2"].reshape(1, 64)
    wh1 = jnp.concatenate([p["psi_rho_w1"], p["psi_e_w1"], p["psi_rhou_w1"]],
                          axis=0).T.astype(bf)
    bh1 = jnp.concatenate([p["psi_rho_b1"], p["psi_e_b1"],
                           p["psi_rhou_b1"]]).reshape(1, 192)
    wh2 = jnp.zeros((192, 4), f32)
    wh2 = wh2.at[0:64, 0].set(p["psi_rho_w2"][0])
    wh2 = wh2.at[64:128, 1].set(p["psi_e_w2"][0])
    wh2 = wh2.at[128:192, 2:4].set(p["psi_rhou_w2"].T)
    wh2 = wh2.astype(bf)
    bh2 = jnp.concatenate([p["psi_rho_b2"], p["psi_e_b2"],
                           p["psi_rhou_b2"]]).reshape(1, 4)

    pad = E_PAD - E_EDGES
    ei_pad = jnp.pad(edge_index, ((0, 0), (0, pad)))
    eaT = jnp.pad(edge_attr.T, ((0, 0), (0, pad)))
    zeros_n = jnp.zeros((N_NODES,), f32)
    node_uT = node_u.T

    g = pl.pallas_call(
        _node_body,
        grid=(N_NODES // BN,),
        in_specs=[
            pl.BlockSpec((BN, 5), lambda i: (i, 0)),
            _full((5, 64)), _full((1, 64)), _full((64, 64)), _full((1, 64)),
            _full((64, 64)), _full((1, 64)), _full((64, 64)), _full((1, 64)),
        ],
        out_specs=pl.BlockSpec((BN, 128), lambda i: (i, 0)),
        out_shape=jax.ShapeDtypeStruct((N_NODES, 128), bf),
    )(node_u, wn1, bn1, wn2, bn2, wg1, bg1, wg2, bg2)

    mesh = plsc.VectorSubcoreMesh(core_axis_name="c", subcore_axis_name="s",
                                  num_cores=NC, num_subcores=NS)

    def gather_half(src_h, dst_h):
        return pl.kernel(
            _gather_body,
            jax.ShapeDtypeStruct((E_H, 128), bf),
            mesh=mesh,
            compiler_params=pltpu.CompilerParams(use_tc_tiling_on_sc=False),
            scratch_types=[
                pltpu.VMEM((GCH,), jnp.int32),
                pltpu.VMEM((GCH,), jnp.int32),
                pltpu.VMEM((GCH,), jnp.int32),
                pltpu.VMEM((GCH,), jnp.int32),
                pltpu.VMEM((GCH, 128), bf),
                pltpu.VMEM((GCH, 128), bf),
                pltpu.VMEM((GCH, 128), bf),
                pltpu.VMEM((GCH, 128), bf),
                pltpu.SemaphoreType.DMA,
                pltpu.SemaphoreType.DMA,
                pltpu.SemaphoreType.DMA,
                pltpu.SemaphoreType.DMA,
            ],
        )(g, src_h, dst_h)

    def edge_half(vg_h, eaT_h, ei_h):
        return pl.pallas_call(
            _edge_body,
            grid=(E_H // BE,),
            in_specs=[
                pl.BlockSpec((BE, 128), lambda i: (i, 0)),
                pl.BlockSpec((3, BE), lambda i: (0, i)),
                pl.BlockSpec((2, BE), lambda i: (0, i)),
                _full((1, 32)), _full((1, 32)), _full((32, 32)), _full((1, 32)),
                _full((128, 128)), _full((32, 128)), _full((1, 128)),
                _full((128, 64)), _full((1, 64)),
                _full((64, 192)), _full((1, 192)), _full((192, 4)), _full((1, 4)),
            ],
            out_specs=[
                pl.BlockSpec((BE // IDXW, 8, IDXW), lambda i: (i, 0, 0)),
                pl.BlockSpec((1, 1), lambda i: (0, 0)),
                pl.BlockSpec((1, 1), lambda i: (0, 0)),
            ],
            out_shape=[
                jax.ShapeDtypeStruct((K8, 8, IDXW), f32),
                jax.ShapeDtypeStruct((1, 1), f32),
                jax.ShapeDtypeStruct((1, 1), f32),
            ],
        )(vg_h, eaT_h, ei_h,
          a_pe, b_pe1, w_pe2, b_pe2, wv, we, b_m1, wm2, b_m2,
          wh1, bh1, wh2, bh2)

    def scatter_half(raw_h, ei3_h):
        return pl.kernel(
            _scatter_body,
            jax.ShapeDtypeStruct((NC, 4, N_NODES), f32),
            mesh=mesh,
            compiler_params=pltpu.CompilerParams(use_tc_tiling_on_sc=False),
            scratch_types=[
                pltpu.VMEM_SHARED((N_NODES,), f32),
                pltpu.VMEM_SHARED((N_NODES,), f32),
                pltpu.VMEM_SHARED((N_NODES,), f32),
                pltpu.VMEM_SHARED((N_NODES,), f32),
                pltpu.VMEM((KR, 8, IDXW), f32),
                pltpu.VMEM((2, KR, IDXW), jnp.int32),
                pltpu.SemaphoreType.DMA,
            ],
        )(raw_h, ei3_h, zeros_n)

    raws, ssums, scnts, sparts = [], [], [], []
    for hh in range(NSPLIT):
        sl = slice(hh * E_H, (hh + 1) * E_H)
        vg_h = gather_half(ei_pad[0, sl], ei_pad[1, sl])
        raw_h, ssum_h, scnt_h = edge_half(vg_h, eaT[:, sl], ei_pad[:, sl])
        raws.append(raw_h)
        ssums.append(ssum_h)
        scnts.append(scnt_h)
        sparts.append(scatter_half(raw_h, ei_pad[:, sl].reshape(2, K8, IDXW)))

    dt = DT_MAX * jax.nn.sigmoid(p["s"])
    dx_est = ((ssums[0][0, 0] + ssums[1][0, 0])
              / (scnts[0][0, 0] + scnts[1][0, 0]))
    scale = (-dt / (dx_est * dx_est)).reshape(1, 1)

    outT = pl.pallas_call(
        _combine_body,
        grid=(1,),
        in_specs=[
            pl.BlockSpec((5, N_NODES), lambda i: (0, 0)),
            pl.BlockSpec((NC, 4, N_NODES), lambda i: (0, 0, 0)),
            pl.BlockSpec((NC, 4, N_NODES), lambda i: (0, 0, 0)),
            pl.BlockSpec(memory_space=pltpu.SMEM),
        ],
        out_specs=pl.BlockSpec((5, N_NODES), lambda i: (0, 0)),
        out_shape=jax.ShapeDtypeStruct((5, N_NODES), f32),
    )(node_uT, sparts[0], sparts[1], scale)

    return outT.T


# speedup vs baseline: 10.1395x; 1.0165x over previous
import jax
import jax.numpy as jnp
from jax import lax
from jax.experimental import pallas as pl
from jax.experimental.pallas import tpu as pltpu
from jax.experimental.pallas import tpu_sc as plsc

N_NODES = 50000
E_EDGES = 800000
DT_MAX = 0.015

NC, NS = 2, 16
NW = NC * NS
E_PAD = 819200
NSPLIT = 2
E_H = E_PAD // NSPLIT
EPT = E_H // NW
GCH = 256
NCH = EPT // GCH
IDXW = 128

K8 = E_H // IDXW
KPT = K8 // NW
KR = 20
SCH = KPT // KR

BN = 5000
BE = 2048

_SQRT1_2 = 0.7071067811865476


def _gelu(x):
    return x * 0.5 * (1.0 + lax.erf(x * _SQRT1_2))


def _node_body(u_ref, wn1, bn1, wn2, bn2, wg1, bg1, wg2, bg2, g_ref):
    x = u_ref[...]
    h = _gelu(jnp.dot(x, wn1[...], preferred_element_type=jnp.float32) + bn1[...])
    h = jnp.dot(h, wn2[...], preferred_element_type=jnp.float32) + bn2[...]
    t = _gelu(jnp.dot(h, wg1[...], preferred_element_type=jnp.float32) + bg1[...])
    g = jnp.dot(t, wg2[...], preferred_element_type=jnp.float32) + bg2[...]
    g_ref[...] = jnp.concatenate([g, jnp.zeros_like(g)], axis=1).astype(jnp.bfloat16)


def _gather_body(g_hbm, src_hbm, dst_hbm, vg_hbm,
                 isrc, idst,
                 buf_a0, buf_b0, buf_a1, buf_b1,
                 gsem0, gsem1, osem0, osem1):
    wid = lax.axis_index("s") * NC + lax.axis_index("c")
    base = wid * EPT
    pltpu.sync_copy(src_hbm.at[pl.ds(base, EPT)], isrc)
    pltpu.sync_copy(dst_hbm.at[pl.ds(base, EPT)], idst)
    sets = [(buf_a0, buf_b0, gsem0, osem0),
            (buf_a1, buf_b1, gsem1, osem1)]

    def fire(st, cc):
        buf_a, buf_b, gsem, _ = st
        for k in range(GCH // IDXW):
            sl = pl.ds(k * IDXW, IDXW)
            isl = pl.ds(cc * GCH + k * IDXW, IDXW)
            pltpu.async_copy(g_hbm.at[isrc.at[isl]], buf_a.at[sl], gsem)
            pltpu.async_copy(g_hbm.at[idst.at[isl]], buf_b.at[sl], gsem)

    def finish(st, cc):
        buf_a, buf_b, gsem, osem = st
        for k in range(GCH // IDXW):
            sl = pl.ds(k * IDXW, IDXW)
            isl = pl.ds(cc * GCH + k * IDXW, IDXW)
            pltpu.make_async_copy(g_hbm.at[isrc.at[isl]], buf_a.at[sl], gsem).wait()
            pltpu.make_async_copy(g_hbm.at[idst.at[isl]], buf_b.at[sl], gsem).wait()

        def add_rows(i, c2):
            for rr in range(4):
                for j in range(2):
                    jl = pl.ds(j * 32, 32)
                    buf_a[i * 4 + rr, jl] = buf_a[i * 4 + rr, jl] + buf_b[i * 4 + rr, jl]
            return c2

        lax.fori_loop(0, GCH // 4, add_rows, 0)
        pltpu.async_copy(buf_a, vg_hbm.at[pl.ds(base + cc * GCH, GCH)], osem)

    def drain_out(st):
        buf_a, _, _, osem = st
        pltpu.make_async_copy(buf_a, vg_hbm.at[pl.ds(base, GCH)], osem).wait()

    fire(sets[0], 0)

    def body(i, carry):
        @pl.when(i > 0)
        def _():
            drain_out(sets[1])
        fire(sets[1], 2 * i + 1)
        finish(sets[0], 2 * i)

        @pl.when(2 * i + 2 < NCH)
        def _():
            drain_out(sets[0])
            fire(sets[0], 2 * i + 2)
        finish(sets[1], 2 * i + 1)
        return carry

    lax.fori_loop(0, NCH // 2, body, 0)
    drain_out(sets[0])
    drain_out(sets[1])


def _edge_body(vg_ref, eaT_ref, ei_ref,
               a_pe, b_pe1, w_pe2, b_pe2,
               wv, we, b_m1, wm2, b_m2,
               wh1, bh1, wh2, bh2,
               raw_ref, ssum_ref, scnt_ref):
    pid = pl.program_id(0)
    vg = vg_ref[...]
    dxR = eaT_ref[0:1, :]
    dyR = eaT_ref[1:2, :]
    rT = eaT_ref[2:3, :]
    r = jnp.transpose(rT)

    bf = jnp.bfloat16
    eps_h = _gelu(r * a_pe[...] + b_pe1[...])
    eps = jnp.dot(eps_h.astype(bf), w_pe2[...],
                  preferred_element_type=jnp.float32) + b_pe2[...]
    m1 = _gelu(jnp.dot(vg, wv[...], preferred_element_type=jnp.float32)
               + jnp.dot(eps.astype(bf), we[...],
                         preferred_element_type=jnp.float32)
               + b_m1[...])
    m = jnp.dot(m1.astype(bf), wm2[...],
                preferred_element_type=jnp.float32) + b_m2[...]
    hh = _gelu(jnp.dot(m.astype(bf), wh1[...],
                       preferred_element_type=jnp.float32) + bh1[...])
    a = jnp.dot(hh.astype(bf), wh2[...],
                preferred_element_type=jnp.float32) + bh2[...]

    aT = jnp.transpose(a)
    invR = 1.0 / (rT + 1e-12)
    nxR = dxR * invR
    nyR = dyR * invR
    n2R = nxR * nxR + nyR * nyR
    maskR = (ei_ref[0:1, :] < ei_ref[1:2, :]).astype(jnp.float32)
    raw0 = aT[0:1, :] * n2R * rT
    raw1 = aT[1:2, :] * n2R * rT
    raw2 = (aT[2:3, :] * nxR - aT[3:4, :] * nyR) * rT
    raw3 = (aT[2:3, :] * nyR + aT[3:4, :] * nxR) * rT
    pos = jnp.concatenate([raw0, raw1, raw2, raw3], axis=0) * maskR
    pos8 = jnp.concatenate([pos, -pos], axis=0)
    for k in range(BE // IDXW):
        raw_ref[k] = pos8[:, k * IDXW:(k + 1) * IDXW]

    ps = jnp.sum(rT * maskR).reshape(1, 1)
    pc = jnp.sum(maskR).reshape(1, 1)

    @pl.when(pid == 0)
    def _():
        ssum_ref[...] = jnp.zeros_like(ssum_ref)
        scnt_ref[...] = jnp.zeros_like(scnt_ref)

    ssum_ref[...] += ps
    scnt_ref[...] += pc


def _scatter_body(raw_hbm, idx_hbm, zeros_hbm, out_hbm,
                  acc0, acc1, acc2, acc3, vbuf, ibuf, ssem):
    c = lax.axis_index("c")
    s = lax.axis_index("s")
    wid = s * NC + c
    accs = [acc0, acc1, acc2, acc3]

    @pl.when(s == 0)
    def _():
        for a in accs:
            pltpu.sync_copy(zeros_hbm, a)

    plsc.subcore_barrier()

    def fire_row(jj):
        for d in range(2):
            for comp in range(4):
                pltpu.async_copy(vbuf.at[jj, comp + 4 * d],
                                 accs[comp].at[ibuf.at[d, jj]], ssem,
                                 add=True)

    def drain_row(jj):
        for d in range(2):
            for comp in range(4):
                pltpu.make_async_copy(vbuf.at[jj, comp + 4 * d],
                                      accs[comp].at[ibuf.at[d, jj]],
                                      ssem).wait()

    def chunk(cc, carry):
        k0 = wid * KPT + cc * KR
        pltpu.sync_copy(raw_hbm.at[pl.ds(k0, KR)], vbuf)
        pltpu.sync_copy(idx_hbm.at[:, pl.ds(k0, KR)], ibuf)
        fire_row(0)

        def row(jj, c2):
            @pl.when(jj + 1 < KR)
            def _():
                fire_row(jj + 1)
            drain_row(jj)
            return c2

        lax.fori_loop(0, KR, row, 0)
        return carry

    lax.fori_loop(0, SCH, chunk, 0)
    plsc.subcore_barrier()

    @pl.when(s == 0)
    def _():
        for comp in range(4):
            pltpu.sync_copy(accs[comp], out_hbm.at[c, comp])


def _combine_body(uT_ref, s0_ref, s1_ref, scale_ref, outT_ref):
    uT = uT_ref[...]
    sc = scale_ref[0, 0]
    d = (s0_ref[0] + s0_ref[1] + s1_ref[0] + s1_ref[1]) * sc
    outT_ref[...] = uT + jnp.concatenate(
        [d[0:2, :], jnp.zeros_like(d[0:1, :]), d[2:4, :]], axis=0)


def _full(shape):
    return pl.BlockSpec(shape, lambda i: tuple(0 for _ in shape))


def kernel(node_u, edge_index, edge_attr, gamma, params):
    p = params
    f32 = jnp.float32
    bf = jnp.bfloat16

    wn1 = p["phi_node_w1"].T
    bn1 = p["phi_node_b1"].reshape(1, 64)
    wn2 = p["phi_node_w2"].T
    bn2 = p["phi_node_b2"].reshape(1, 64)
    wg1 = jnp.concatenate([p["phi1_w1"], p["phi2_w1"]], axis=0).T
    bg1 = jnp.concatenate([p["phi1_b1"], p["phi2_b1"]]).reshape(1, 64)
    wg2 = jnp.concatenate([p["phi1_w2"].T, p["phi2_w2"].T], axis=0)
    bg2 = (p["phi1_b2"] + p["phi2_b2"]).reshape(1, 64)

    a_pe = p["phi_edge_w1"][:, 0].reshape(1, 32)
    b_pe1 = p["phi_edge_b1"].reshape(1, 32)
    w_pe2 = p["phi_edge_w2"].T.astype(bf)
    b_pe2 = p["phi_edge_b2"].reshape(1, 32)
    wv = jnp.concatenate([p["phi_msg_w1"][:, :64].T,
                          jnp.zeros((64, 128), f32)], axis=0).astype(bf)
    we = p["phi_msg_w1"][:, 64:].T.astype(bf)
    b_m1 = p["phi_msg_b1"].reshape(1, 128)
    wm2 = p["phi_msg_w2"].T.astype(bf)
    b_m2 = p["phi_msg_b2"].reshape(1, 64)
    wh1 = jnp.concatenate([p["psi_rho_w1"], p["psi_e_w1"], p["psi_rhou_w1"]],
                          axis=0).T.astype(bf)
    bh1 = jnp.concatenate([p["psi_rho_b1"], p["psi_e_b1"],
                           p["psi_rhou_b1"]]).reshape(1, 192)
    wh2 = jnp.zeros((192, 4), f32)
    wh2 = wh2.at[0:64, 0].set(p["psi_rho_w2"][0])
    wh2 = wh2.at[64:128, 1].set(p["psi_e_w2"][0])
    wh2 = wh2.at[128:192, 2:4].set(p["psi_rhou_w2"].T)
    wh2 = wh2.astype(bf)
    bh2 = jnp.concatenate([p["psi_rho_b2"], p["psi_e_b2"],
                           p["psi_rhou_b2"]]).reshape(1, 4)

    pad = E_PAD - E_EDGES
    ei_pad = jnp.pad(edge_index, ((0, 0), (0, pad)))
    eaT = jnp.pad(edge_attr.T, ((0, 0), (0, pad)))
    zeros_n = jnp.zeros((N_NODES,), f32)
    node_uT = node_u.T

    g = pl.pallas_call(
        _node_body,
        grid=(N_NODES // BN,),
        in_specs=[
            pl.BlockSpec((BN, 5), lambda i: (i, 0)),
            _full((5, 64)), _full((1, 64)), _full((64, 64)), _full((1, 64)),
            _full((64, 64)), _full((1, 64)), _full((64, 64)), _full((1, 64)),
        ],
        out_specs=pl.BlockSpec((BN, 128), lambda i: (i, 0)),
        out_shape=jax.ShapeDtypeStruct((N_NODES, 128), bf),
    )(node_u, wn1, bn1, wn2, bn2, wg1, bg1, wg2, bg2)

    mesh = plsc.VectorSubcoreMesh(core_axis_name="c", subcore_axis_name="s",
                                  num_cores=NC, num_subcores=NS)

    def gather_half(src_h, dst_h):
        return pl.kernel(
            _gather_body,
            jax.ShapeDtypeStruct((E_H, 128), bf),
            mesh=mesh,
            compiler_params=pltpu.CompilerParams(use_tc_tiling_on_sc=False),
            scratch_types=[
                pltpu.VMEM((EPT,), jnp.int32),
                pltpu.VMEM((EPT,), jnp.int32),
                pltpu.VMEM((GCH, 128), bf),
                pltpu.VMEM((GCH, 128), bf),
                pltpu.VMEM((GCH, 128), bf),
                pltpu.VMEM((GCH, 128), bf),
                pltpu.SemaphoreType.DMA,
                pltpu.SemaphoreType.DMA,
                pltpu.SemaphoreType.DMA,
                pltpu.SemaphoreType.DMA,
            ],
        )(g, src_h, dst_h)

    def edge_half(vg_h, eaT_h, ei_h):
        return pl.pallas_call(
            _edge_body,
            grid=(E_H // BE,),
            in_specs=[
                pl.BlockSpec((BE, 128), lambda i: (i, 0)),
                pl.BlockSpec((3, BE), lambda i: (0, i)),
                pl.BlockSpec((2, BE), lambda i: (0, i)),
                _full((1, 32)), _full((1, 32)), _full((32, 32)), _full((1, 32)),
                _full((128, 128)), _full((32, 128)), _full((1, 128)),
                _full((128, 64)), _full((1, 64)),
                _full((64, 192)), _full((1, 192)), _full((192, 4)), _full((1, 4)),
            ],
            out_specs=[
                pl.BlockSpec((BE // IDXW, 8, IDXW), lambda i: (i, 0, 0)),
                pl.BlockSpec((1, 1), lambda i: (0, 0)),
                pl.BlockSpec((1, 1), lambda i: (0, 0)),
            ],
            out_shape=[
                jax.ShapeDtypeStruct((K8, 8, IDXW), f32),
                jax.ShapeDtypeStruct((1, 1), f32),
                jax.ShapeDtypeStruct((1, 1), f32),
            ],
        )(vg_h, eaT_h, ei_h,
          a_pe, b_pe1, w_pe2, b_pe2, wv, we, b_m1, wm2, b_m2,
          wh1, bh1, wh2, bh2)

    def scatter_half(raw_h, ei3_h):
        return pl.kernel(
            _scatter_body,
            jax.ShapeDtypeStruct((NC, 4, N_NODES), f32),
            mesh=mesh,
            compiler_params=pltpu.CompilerParams(use_tc_tiling_on_sc=False),
            scratch_types=[
                pltpu.VMEM_SHARED((N_NODES,), f32),
                pltpu.VMEM_SHARED((N_NODES,), f32),
                pltpu.VMEM_SHARED((N_NODES,), f32),
                pltpu.VMEM_SHARED((N_NODES,), f32),
                pltpu.VMEM((KR, 8, IDXW), f32),
                pltpu.VMEM((2, KR, IDXW), jnp.int32),
                pltpu.SemaphoreType.DMA,
            ],
        )(raw_h, ei3_h, zeros_n)

    raws, ssums, scnts, sparts = [], [], [], []
    for hh in range(NSPLIT):
        sl = slice(hh * E_H, (hh + 1) * E_H)
        vg_h = gather_half(ei_pad[0, sl], ei_pad[1, sl])
        raw_h, ssum_h, scnt_h = edge_half(vg_h, eaT[:, sl], ei_pad[:, sl])
        raws.append(raw_h)
        ssums.append(ssum_h)
        scnts.append(scnt_h)
        sparts.append(scatter_half(raw_h, ei_pad[:, sl].reshape(2, K8, IDXW)))

    dt = DT_MAX * jax.nn.sigmoid(p["s"])
    dx_est = ((ssums[0][0, 0] + ssums[1][0, 0])
              / (scnts[0][0, 0] + scnts[1][0, 0]))
    scale = (-dt / (dx_est * dx_est)).reshape(1, 1)

    outT = pl.pallas_call(
        _combine_body,
        grid=(1,),
        in_specs=[
            pl.BlockSpec((5, N_NODES), lambda i: (0, 0)),
            pl.BlockSpec((NC, 4, N_NODES), lambda i: (0, 0, 0)),
            pl.BlockSpec((NC, 4, N_NODES), lambda i: (0, 0, 0)),
            pl.BlockSpec(memory_space=pltpu.SMEM),
        ],
        out_specs=pl.BlockSpec((5, N_NODES), lambda i: (0, 0)),
        out_shape=jax.ShapeDtypeStruct((5, N_NODES), f32),
    )(node_uT, sparts[0], sparts[1], scale)

    return outT.T
